# baseline (speedup 1.0000x reference)
import jax
import jax.numpy as jnp
from jax import lax
from jax.experimental import pallas as pl
from jax.experimental.pallas import tpu as pltpu
from jax.experimental.pallas import tpu_sc as plsc

N = 10000
D = 128
H = 2
C = 256
F = 128

NS = 16
E_TOT = 320000 + N
CHUNK = 64
G = 8
NGROUPS = 42
CHUNKS = G * NGROUPS
CHUNKS_P = CHUNKS + G
PER_TILE = CHUNKS * CHUNK
EP = NS * PER_TILE
STRIPE = 640
NPAD = NS * STRIPE


def _prep_body(x_ref, ws_ref, wd_ref, asrc_ref, adst_ref, wl_ref,
               y_ref, a_ref):
    xb = x_ref[...]
    m0 = jnp.dot(ws_ref[:, :C], wl_ref[:C, :],
                 preferred_element_type=jnp.float32)
    m1 = jnp.dot(ws_ref[:, C:], wl_ref[C:, :],
                 preferred_element_type=jnp.float32)
    y_ref[0] = jnp.dot(xb, m0, preferred_element_type=jnp.float32)
    y_ref[1] = jnp.dot(xb, m1, preferred_element_type=jnp.float32)
    vs0 = jnp.sum(ws_ref[:, :C] * asrc_ref[0, 0][None, :], axis=1)
    vs1 = jnp.sum(ws_ref[:, C:] * asrc_ref[0, 1][None, :], axis=1)
    vd0 = jnp.sum(wd_ref[:, :C] * adst_ref[0, 0][None, :], axis=1)
    vd1 = jnp.sum(wd_ref[:, C:] * adst_ref[0, 1][None, :], axis=1)
    v = jnp.stack([vs0, vs1, vd0, vd1], axis=1)
    a_ref[...] = jnp.dot(xb, v, preferred_element_type=jnp.float32)


def _finish_body(acc_ref, den_ref, bc_ref, wl_ref, bl_ref, out_ref):
    den = den_ref[...] + 1e-16
    bias = (jnp.dot(bc_ref[...][None, :], wl_ref[...],
                    preferred_element_type=jnp.float32)[0] + bl_ref[...])
    out_ref[...] = (acc_ref[0] / den[0][:, None]
                    + acc_ref[1] / den[1][:, None]
                    + bias[None, :])


BB = 4


def _sc_agg(y_hbm, a_hbm, src_hbm, dst_hbm, out_hbm, den_hbm,
            srcg, dstg, dvi, avb, bvb, ybuf, wbuf, zbuf, acc_sh, den_sh,
            sem_y0, sem_y1, sem_y2, sem_y3,
            sem_a0, sem_a1, sem_a2, sem_a3,
            sem_b0, sem_b1, sem_b2, sem_b3,
            sem_acc0, sem_acc1, sem_acc2, sem_acc3,
            sem_den0, sem_den1, sem_den2, sem_den3):
    c = lax.axis_index("c")
    s = lax.axis_index("s")
    doff = (2 + c) * N
    sem_y = (sem_y0, sem_y1, sem_y2, sem_y3)
    sem_a = (sem_a0, sem_a1, sem_a2, sem_a3)
    sem_b = (sem_b0, sem_b1, sem_b2, sem_b3)
    sem_acc = (sem_acc0, sem_acc1, sem_acc2, sem_acc3)
    sem_den = (sem_den0, sem_den1, sem_den2, sem_den3)

    def _zy(i, _):
        for j in range(F // 16):
            ybuf[0, i, pl.ds(j * 16, 16)] = jnp.zeros((16,), jnp.float32)
        return 0
    lax.fori_loop(0, CHUNK, _zy, 0)
    for j in range(STRIPE // 16):
        zbuf[pl.ds(j * 16, 16)] = jnp.zeros((16,), jnp.float32)
    pltpu.sync_copy(zbuf, den_sh.at[pl.ds(s * STRIPE, STRIPE)])
    for k in range(STRIPE // CHUNK):
        pltpu.sync_copy(ybuf.at[0],
                        acc_sh.at[pl.ds(s * STRIPE + k * CHUNK, CHUNK)])
    plsc.subcore_barrier()

    def _group(g, _):
        pltpu.sync_copy(src_hbm.at[c].at[s].at[pl.ds(g * G, G)], srcg)
        pltpu.sync_copy(dst_hbm.at[c].at[s].at[pl.ds(g * G, G)], dstg)

        def _launch(ch, bb):
            for i in range(CHUNK // 16):
                sl = pl.ds(i * 16, 16)
                dvi[bb, sl] = dstg[ch, sl] + doff
            return (
                pltpu.async_copy(y_hbm.at[srcg.at[ch]],
                                 ybuf.at[bb], sem_y[bb]),
                pltpu.async_copy(a_hbm.at[srcg.at[ch]],
                                 avb.at[bb], sem_a[bb]),
                pltpu.async_copy(a_hbm.at[dvi.at[bb]],
                                 bvb.at[bb], sem_b[bb]),
            )

        gathers = [None] * BB
        scatters = [None] * BB
        gathers[0] = _launch(0, 0)
        gathers[1] = _launch(1, 1)

        for ch in range(G):
            bb = ch % BB
            nb = (ch + 2) % BB
            if scatters[nb] is not None:
                scatters[nb][0].wait()
                scatters[nb][1].wait()
                scatters[nb] = None
            if ch + 2 < G:
                gathers[nb] = _launch(ch + 2, nb)
            gathers[bb][1].wait()
            gathers[bb][2].wait()
            gbase = s * PER_TILE + (g * G + ch) * CHUNK
            for i in range(CHUNK // 16):
                sl = pl.ds(i * 16, 16)
                t = avb[bb, sl] + bvb[bb, sl]
                t = jnp.where(t > 0, t, 0.2 * t)
                w = jnp.exp(t)
                gid = gbase + i * 16 + lax.iota(jnp.int32, 16)
                w = jnp.where(gid < E_TOT, w, 0.0)
                wbuf[bb, sl] = w
            gathers[bb][0].wait()

            def _scale(k, _):
                w16 = wbuf[bb, pl.ds(k * 16, 16)]
                for l in range(16):
                    e = k * 16 + l
                    wv = jnp.full((16,), w16[l])
                    for j in range(F // 16):
                        sl2 = pl.ds(j * 16, 16)
                        ybuf[bb, e, sl2] = ybuf[bb, e, sl2] * wv
                return 0
            lax.fori_loop(0, CHUNK // 16, _scale, 0)

            scatters[bb] = (
                pltpu.async_copy(ybuf.at[bb], acc_sh.at[dstg.at[ch]],
                                 sem_acc[bb], add=True),
                pltpu.async_copy(wbuf.at[bb].at[pl.ds(0, CHUNK)],
                                 den_sh.at[dstg.at[ch]],
                                 sem_den[bb], add=True),
            )
        for q in range(3):
            if scatters[q] is not None:
                scatters[q][0].wait()
                scatters[q][1].wait()
        return 0
    lax.fori_loop(0, NGROUPS, _group, 0)
    plsc.subcore_barrier()

    pltpu.sync_copy(acc_sh.at[pl.ds(s * STRIPE, STRIPE)],
                    out_hbm.at[c].at[s])
    pltpu.sync_copy(den_sh.at[pl.ds(s * STRIPE, STRIPE)],
                    den_hbm.at[c].at[s])


def kernel(x, edge_index, edge_attr, Wsrc, Wdst, att_src, att_dst,
           b_conv, W_lin, b_lin):
    bN = 2000
    grid = N // bN

    y, a = pl.pallas_call(
        _prep_body,
        grid=(grid,),
        in_specs=[
            pl.BlockSpec((bN, D), lambda i: (i, 0)),
            pl.BlockSpec((D, H * C), lambda i: (0, 0)),
            pl.BlockSpec((D, H * C), lambda i: (0, 0)),
            pl.BlockSpec((1, H, C), lambda i: (0, 0, 0)),
            pl.BlockSpec((1, H, C), lambda i: (0, 0, 0)),
            pl.BlockSpec((H * C, F), lambda i: (0, 0)),
        ],
        out_specs=[
            pl.BlockSpec((H, bN, F), lambda i: (0, i, 0)),
            pl.BlockSpec((bN, 4), lambda i: (i, 0)),
        ],
        out_shape=[
            jax.ShapeDtypeStruct((H, N, F), jnp.float32),
            jax.ShapeDtypeStruct((N, 4), jnp.float32),
        ],
    )(x, Wsrc, Wdst, att_src, att_dst, W_lin)

    y2 = y.reshape(H * N, F)
    a4 = a.T.reshape(4 * N)

    loop = jnp.arange(N, dtype=jnp.int32)
    pad = jnp.zeros((EP - E_TOT,), jnp.int32)
    gpad = jnp.zeros((NS, G, CHUNK), jnp.int32)
    src3 = jnp.concatenate(
        [jnp.concatenate([edge_index[0], loop, pad]).reshape(NS, CHUNKS, CHUNK),
         gpad], axis=1)
    dst3 = jnp.concatenate(
        [jnp.concatenate([edge_index[1], loop, pad]).reshape(NS, CHUNKS, CHUNK),
         gpad], axis=1)
    src4 = jnp.stack([src3, src3 + N])
    dst4 = jnp.stack([dst3, dst3])

    mesh = plsc.VectorSubcoreMesh(core_axis_name="c", subcore_axis_name="s")
    acc, den = pl.kernel(
        _sc_agg,
        mesh=mesh,
        compiler_params=pltpu.CompilerParams(needs_layout_passes=False),
        out_type=[
            jax.ShapeDtypeStruct((H, NS, STRIPE, F), jnp.float32),
            jax.ShapeDtypeStruct((H, NS, STRIPE), jnp.float32),
        ],
        scratch_types=[
            pltpu.VMEM((G, CHUNK), jnp.int32),
            pltpu.VMEM((G, CHUNK), jnp.int32),
            pltpu.VMEM((BB, CHUNK), jnp.int32),
            pltpu.VMEM((BB, CHUNK), jnp.float32),
            pltpu.VMEM((BB, CHUNK), jnp.float32),
            pltpu.VMEM((BB, CHUNK, F), jnp.float32),
            pltpu.VMEM((BB, CHUNK + 16), jnp.float32),
            pltpu.VMEM((STRIPE,), jnp.float32),
            pltpu.VMEM_SHARED((NPAD, F), jnp.float32),
            pltpu.VMEM_SHARED((NPAD,), jnp.float32),
        ] + [pltpu.SemaphoreType.DMA] * 20,
    )(y2, a4, src4, dst4)
    acc = acc.reshape(H, NPAD, F)[:, :N]
    den2 = den.reshape(H, NPAD)[:, :N]

    out = pl.pallas_call(
        _finish_body,
        out_shape=jax.ShapeDtypeStruct((N, F), jnp.float32),
    )(acc, den2, b_conv, W_lin, b_lin)
    return out


# speedup vs baseline: 1.3967x; 1.3967x over previous
import jax
import jax.numpy as jnp
from jax import lax
from jax.experimental import pallas as pl
from jax.experimental.pallas import tpu as pltpu
from jax.experimental.pallas import tpu_sc as plsc

N = 10000
D = 128
H = 2
C = 256
F = 128

NS = 16
E_TOT = 320000 + N
CHUNK = 80
G = 8
NGROUPS = 33
CHUNKS = G * NGROUPS
PER_TILE = CHUNKS * CHUNK
EP = NS * PER_TILE
STRIPE = 640
NPAD = NS * STRIPE


def _prep_body(x_ref, ws_ref, wd_ref, asrc_ref, adst_ref, wl_ref,
               y_ref, a_ref):
    xb = x_ref[...]
    m0 = jnp.dot(ws_ref[:, :C], wl_ref[:C, :],
                 preferred_element_type=jnp.float32)
    m1 = jnp.dot(ws_ref[:, C:], wl_ref[C:, :],
                 preferred_element_type=jnp.float32)
    y_ref[0] = jnp.dot(xb, m0, preferred_element_type=jnp.float32)
    y_ref[1] = jnp.dot(xb, m1, preferred_element_type=jnp.float32)
    vs0 = jnp.sum(ws_ref[:, :C] * asrc_ref[0, 0][None, :], axis=1)
    vs1 = jnp.sum(ws_ref[:, C:] * asrc_ref[0, 1][None, :], axis=1)
    vd0 = jnp.sum(wd_ref[:, :C] * adst_ref[0, 0][None, :], axis=1)
    vd1 = jnp.sum(wd_ref[:, C:] * adst_ref[0, 1][None, :], axis=1)
    v = jnp.stack([vs0, vs1, vd0, vd1], axis=1)
    a_ref[...] = jnp.dot(xb, v, preferred_element_type=jnp.float32)


def _finish_body(acc_ref, den_ref, bc_ref, wl_ref, bl_ref, out_ref):
    den = den_ref[...] + 1e-16
    bias = (jnp.dot(bc_ref[...][None, :], wl_ref[...],
                    preferred_element_type=jnp.float32)[0] + bl_ref[...])
    out_ref[...] = (acc_ref[0] / den[0][:, None]
                    + acc_ref[1] / den[1][:, None]
                    + bias[None, :])


def _sc_agg(y_hbm, a_hbm, src_hbm, dst_hbm, out_hbm, den_hbm,
            a_v, srcg, dstg, ybuf, wbuf, zbuf, acc_sh, den_sh,
            sem_y0, sem_y1, sem_acc0, sem_acc1, sem_den0, sem_den1):
    c = lax.axis_index("c")
    s = lax.axis_index("s")
    sem_y = (sem_y0, sem_y1)
    sem_acc = (sem_acc0, sem_acc1)
    sem_den = (sem_den0, sem_den1)
    coff = c * N

    pltpu.sync_copy(a_hbm.at[c], a_v)

    def _zy(i, _):
        for j in range(F // 16):
            ybuf[0, i, pl.ds(j * 16, 16)] = jnp.zeros((16,), jnp.float32)
        return 0
    lax.fori_loop(0, CHUNK, _zy, 0)
    for j in range(STRIPE // 16):
        zbuf[pl.ds(j * 16, 16)] = jnp.zeros((16,), jnp.float32)
    pltpu.sync_copy(zbuf, den_sh.at[pl.ds(s * STRIPE, STRIPE)])
    for k in range(STRIPE // CHUNK):
        pltpu.sync_copy(ybuf.at[0],
                        acc_sh.at[pl.ds(s * STRIPE + k * CHUNK, CHUNK)])
    plsc.subcore_barrier()

    def _group(g, _):
        pltpu.sync_copy(src_hbm.at[c].at[s].at[pl.ds(g * G, G)], srcg)
        pltpu.sync_copy(dst_hbm.at[c].at[s].at[pl.ds(g * G, G)], dstg)

        def _launch(ch, bb):
            return pltpu.async_copy(y_hbm.at[srcg.at[ch]],
                                    ybuf.at[bb], sem_y[bb])

        gathers = [None, None]
        scatters = [None, None]
        gathers[0] = _launch(0, 0)

        for ch in range(G):
            bb = ch % 2
            nb = 1 - bb
            if scatters[nb] is not None:
                scatters[nb][0].wait()
                scatters[nb][1].wait()
                scatters[nb] = None
            if ch + 1 < G:
                gathers[nb] = _launch(ch + 1, nb)
            gbase = s * PER_TILE + (g * G + ch) * CHUNK
            for i in range(CHUNK // 16):
                sl = pl.ds(i * 16, 16)
                s16 = srcg[ch, sl] - coff
                d16 = dstg[ch, sl]
                av = plsc.load_gather(a_v, [s16])
                bv = plsc.load_gather(a_v, [d16 + N])
                t = av + bv
                t = jnp.where(t > 0, t, 0.2 * t)
                w = jnp.exp(t)
                gid = gbase + i * 16 + lax.iota(jnp.int32, 16)
                w = jnp.where(gid < E_TOT, w, 0.0)
                wbuf[bb, sl] = w
            gathers[bb].wait()

            def _scale(k, _):
                w16 = wbuf[bb, pl.ds(k * 16, 16)]
                for l in range(16):
                    e = k * 16 + l
                    wv = jnp.full((16,), w16[l])
                    for j in range(F // 16):
                        sl2 = pl.ds(j * 16, 16)
                        ybuf[bb, e, sl2] = ybuf[bb, e, sl2] * wv
                return 0
            lax.fori_loop(0, CHUNK // 16, _scale, 0)

            scatters[bb] = (
                pltpu.async_copy(ybuf.at[bb], acc_sh.at[dstg.at[ch]],
                                 sem_acc[bb], add=True),
                pltpu.async_copy(wbuf.at[bb].at[pl.ds(0, CHUNK)],
                                 den_sh.at[dstg.at[ch]],
                                 sem_den[bb], add=True),
            )
        for q in range(2):
            if scatters[q] is not None:
                scatters[q][0].wait()
                scatters[q][1].wait()
        return 0
    lax.fori_loop(0, NGROUPS, _group, 0)
    plsc.subcore_barrier()

    pltpu.sync_copy(acc_sh.at[pl.ds(s * STRIPE, STRIPE)],
                    out_hbm.at[c].at[s])
    pltpu.sync_copy(den_sh.at[pl.ds(s * STRIPE, STRIPE)],
                    den_hbm.at[c].at[s])


def kernel(x, edge_index, edge_attr, Wsrc, Wdst, att_src, att_dst,
           b_conv, W_lin, b_lin):
    bN = 2000
    grid = N // bN

    y, a = pl.pallas_call(
        _prep_body,
        grid=(grid,),
        in_specs=[
            pl.BlockSpec((bN, D), lambda i: (i, 0)),
            pl.BlockSpec((D, H * C), lambda i: (0, 0)),
            pl.BlockSpec((D, H * C), lambda i: (0, 0)),
            pl.BlockSpec((1, H, C), lambda i: (0, 0, 0)),
            pl.BlockSpec((1, H, C), lambda i: (0, 0, 0)),
            pl.BlockSpec((H * C, F), lambda i: (0, 0)),
        ],
        out_specs=[
            pl.BlockSpec((H, bN, F), lambda i: (0, i, 0)),
            pl.BlockSpec((bN, 4), lambda i: (i, 0)),
        ],
        out_shape=[
            jax.ShapeDtypeStruct((H, N, F), jnp.float32),
            jax.ShapeDtypeStruct((N, 4), jnp.float32),
        ],
    )(x, Wsrc, Wdst, att_src, att_dst, W_lin)

    y2 = y.reshape(H * N, F)
    a2 = jnp.stack([
        jnp.concatenate([a[:, 0], a[:, 2]]),
        jnp.concatenate([a[:, 1], a[:, 3]]),
    ])

    loop = jnp.arange(N, dtype=jnp.int32)
    pad = jnp.zeros((EP - E_TOT,), jnp.int32)
    src3 = jnp.concatenate([edge_index[0], loop, pad]).reshape(
        NS, CHUNKS, CHUNK)
    dst3 = jnp.concatenate([edge_index[1], loop, pad]).reshape(
        NS, CHUNKS, CHUNK)
    src4 = jnp.stack([src3, src3 + N])
    dst4 = jnp.stack([dst3, dst3])

    mesh = plsc.VectorSubcoreMesh(core_axis_name="c", subcore_axis_name="s")
    acc, den = pl.kernel(
        _sc_agg,
        mesh=mesh,
        compiler_params=pltpu.CompilerParams(needs_layout_passes=False),
        out_type=[
            jax.ShapeDtypeStruct((H, NS, STRIPE, F), jnp.float32),
            jax.ShapeDtypeStruct((H, NS, STRIPE), jnp.float32),
        ],
        scratch_types=[
            pltpu.VMEM((2 * N,), jnp.float32),
            pltpu.VMEM((G, CHUNK), jnp.int32),
            pltpu.VMEM((G, CHUNK), jnp.int32),
            pltpu.VMEM((2, CHUNK, F), jnp.float32),
            pltpu.VMEM((2, CHUNK + 16), jnp.float32),
            pltpu.VMEM((STRIPE,), jnp.float32),
            pltpu.VMEM_SHARED((NPAD, F), jnp.float32),
            pltpu.VMEM_SHARED((NPAD,), jnp.float32),
        ] + [pltpu.SemaphoreType.DMA] * 6,
    )(y2, a2, src4, dst4)
    acc = acc.reshape(H, NPAD, F)[:, :N]
    den2 = den.reshape(H, NPAD)[:, :N]

    out = pl.pallas_call(
        _finish_body,
        out_shape=jax.ShapeDtypeStruct((N, F), jnp.float32),
    )(acc, den2, b_conv, W_lin, b_lin)
    return out


# speedup vs baseline: 2.3640x; 1.6926x over previous
import jax
import jax.numpy as jnp
from jax import lax
from jax.experimental import pallas as pl
from jax.experimental.pallas import tpu as pltpu
from jax.experimental.pallas import tpu_sc as plsc

N = 10000
D = 128
H = 2
C = 256
F = 128

NS = 16
E_TOT = 320000 + N
CHUNK = 96
G = 8
NGROUPS = 27
CHUNKS = G * NGROUPS
PER_TILE = CHUNKS * CHUNK
EP = NS * PER_TILE
STRIPE = 632
NPAD = NS * STRIPE


def _prep_body(x_ref, ws_ref, wd_ref, asrc_ref, adst_ref, wl_ref,
               y_ref, a_ref):
    xb = x_ref[...]
    m0 = jnp.dot(ws_ref[:, :C], wl_ref[:C, :],
                 preferred_element_type=jnp.float32)
    m1 = jnp.dot(ws_ref[:, C:], wl_ref[C:, :],
                 preferred_element_type=jnp.float32)
    y_ref[0] = jnp.dot(xb, m0, preferred_element_type=jnp.float32)
    y_ref[1] = jnp.dot(xb, m1, preferred_element_type=jnp.float32)
    vs0 = jnp.sum(ws_ref[:, :C] * asrc_ref[0, 0][None, :], axis=1)
    vs1 = jnp.sum(ws_ref[:, C:] * asrc_ref[0, 1][None, :], axis=1)
    vd0 = jnp.sum(wd_ref[:, :C] * adst_ref[0, 0][None, :], axis=1)
    vd1 = jnp.sum(wd_ref[:, C:] * adst_ref[0, 1][None, :], axis=1)
    v = jnp.stack([vs0, vs1, vd0, vd1], axis=1)
    a_ref[...] = jnp.dot(xb, v, preferred_element_type=jnp.float32)


def _finish_body(acc_ref, den_ref, bc_ref, wl_ref, bl_ref, out_ref):
    den = den_ref[...] + 1e-16
    bias = (jnp.dot(bc_ref[...][None, :], wl_ref[...],
                    preferred_element_type=jnp.float32)[0] + bl_ref[...])
    out_ref[...] = (acc_ref[0] / den[0][:, None]
                    + acc_ref[1] / den[1][:, None]
                    + bias[None, :])


def _sc_agg(y_hbm, a_hbm, src_hbm, dst_hbm, out_hbm, den_hbm,
            a_v, srcg, dstg, ybuf, wbuf, zbuf, acc_sh, den_sh,
            sem_y0, sem_y1, sem_acc0, sem_acc1, sem_den0, sem_den1):
    c = lax.axis_index("c")
    s = lax.axis_index("s")
    sem_y = (sem_y0, sem_y1)
    sem_acc = (sem_acc0, sem_acc1)
    sem_den = (sem_den0, sem_den1)
    coff = c * N

    pltpu.sync_copy(a_hbm.at[c], a_v)

    def _zy(i, _):
        for j in range(F // 16):
            ybuf[0, i, pl.ds(j * 16, 16)] = jnp.zeros((16,), jnp.float32)
        return 0
    lax.fori_loop(0, CHUNK, _zy, 0)
    for j in range(640 // 16):
        zbuf[pl.ds(j * 16, 16)] = jnp.zeros((16,), jnp.float32)
    pltpu.sync_copy(zbuf, den_sh.at[pl.ds(s * 640, 640)])
    for k in range(STRIPE // CHUNK):
        pltpu.sync_copy(ybuf.at[0],
                        acc_sh.at[pl.ds(s * STRIPE + k * CHUNK, CHUNK)])
    _rem = STRIPE % CHUNK
    if _rem:
        pltpu.sync_copy(
            ybuf.at[0].at[pl.ds(0, _rem)],
            acc_sh.at[pl.ds(s * STRIPE + (STRIPE // CHUNK) * CHUNK, _rem)])
    plsc.subcore_barrier()

    def _group(g, _):
        pltpu.sync_copy(src_hbm.at[c].at[s].at[pl.ds(g * G, G)], srcg)
        pltpu.sync_copy(dst_hbm.at[c].at[s].at[pl.ds(g * G, G)], dstg)

        def _launch(ch, bb):
            return pltpu.async_copy(y_hbm.at[srcg.at[ch]],
                                    ybuf.at[bb], sem_y[bb])

        gathers = [None, None]
        scatters = [None, None]
        gathers[0] = _launch(0, 0)

        for ch in range(G):
            bb = ch % 2
            nb = 1 - bb
            if scatters[nb] is not None:
                scatters[nb][0].wait()
                scatters[nb][1].wait()
                scatters[nb] = None
            if ch + 1 < G:
                gathers[nb] = _launch(ch + 1, nb)
            gbase = s * PER_TILE + (g * G + ch) * CHUNK
            for i in range(CHUNK // 16):
                sl = pl.ds(i * 16, 16)
                s16 = srcg[ch, sl] - coff
                d16 = dstg[ch, sl]
                av = plsc.load_gather(a_v, [s16])
                bv = plsc.load_gather(a_v, [d16 + N])
                t = av + bv
                t = jnp.where(t > 0, t, 0.2 * t)
                w = jnp.exp(t)
                gid = gbase + i * 16 + lax.iota(jnp.int32, 16)
                w = jnp.where(gid < E_TOT, w, 0.0)
                wbuf[bb, sl] = w
            gathers[bb].wait()

            def _scale(k, _):
                w16 = wbuf[bb, pl.ds(k * 16, 16)]
                for l in range(16):
                    e = k * 16 + l
                    wv = jnp.full((16,), w16[l])
                    for j in range(F // 16):
                        sl2 = pl.ds(j * 16, 16)
                        ybuf[bb, e, sl2] = ybuf[bb, e, sl2] * wv
                return 0
            lax.fori_loop(0, CHUNK // 16, _scale, 0)

            scatters[bb] = (
                pltpu.async_copy(ybuf.at[bb], acc_sh.at[dstg.at[ch]],
                                 sem_acc[bb], add=True),
                pltpu.async_copy(wbuf.at[bb].at[pl.ds(0, CHUNK)],
                                 den_sh.at[dstg.at[ch]],
                                 sem_den[bb], add=True),
            )
        for q in range(2):
            if scatters[q] is not None:
                scatters[q][0].wait()
                scatters[q][1].wait()
        return 0
    lax.fori_loop(0, NGROUPS, _group, 0)
    plsc.subcore_barrier()

    pltpu.sync_copy(acc_sh.at[pl.ds(s * STRIPE, STRIPE)],
                    out_hbm.at[c].at[s])
    pltpu.sync_copy(den_sh.at[pl.ds(s * 640, 640)],
                    den_hbm.at[c].at[s])


def kernel(x, edge_index, edge_attr, Wsrc, Wdst, att_src, att_dst,
           b_conv, W_lin, b_lin):
    bN = 2000
    grid = N // bN

    y, a = pl.pallas_call(
        _prep_body,
        grid=(grid,),
        in_specs=[
            pl.BlockSpec((bN, D), lambda i: (i, 0)),
            pl.BlockSpec((D, H * C), lambda i: (0, 0)),
            pl.BlockSpec((D, H * C), lambda i: (0, 0)),
            pl.BlockSpec((1, H, C), lambda i: (0, 0, 0)),
            pl.BlockSpec((1, H, C), lambda i: (0, 0, 0)),
            pl.BlockSpec((H * C, F), lambda i: (0, 0)),
        ],
        out_specs=[
            pl.BlockSpec((H, bN, F), lambda i: (0, i, 0)),
            pl.BlockSpec((bN, 4), lambda i: (i, 0)),
        ],
        out_shape=[
            jax.ShapeDtypeStruct((H, N, F), jnp.float32),
            jax.ShapeDtypeStruct((N, 4), jnp.float32),
        ],
    )(x, Wsrc, Wdst, att_src, att_dst, W_lin)

    y2 = y.reshape(H * N, F)
    a2 = jnp.stack([
        jnp.concatenate([a[:, 0], a[:, 2]]),
        jnp.concatenate([a[:, 1], a[:, 3]]),
    ])

    loop = jnp.arange(N, dtype=jnp.int32)
    pad = jnp.zeros((EP - E_TOT,), jnp.int32)
    src3 = jnp.concatenate([edge_index[0], loop, pad]).reshape(
        NS, CHUNKS, CHUNK)
    dst3 = jnp.concatenate([edge_index[1], loop, pad]).reshape(
        NS, CHUNKS, CHUNK)
    src4 = jnp.stack([src3, src3 + N])
    dst4 = jnp.stack([dst3, dst3])

    mesh = plsc.VectorSubcoreMesh(core_axis_name="c", subcore_axis_name="s")
    acc, den = pl.kernel(
        _sc_agg,
        mesh=mesh,
        compiler_params=pltpu.CompilerParams(needs_layout_passes=False),
        out_type=[
            jax.ShapeDtypeStruct((H, NS, STRIPE, F), jnp.float32),
            jax.ShapeDtypeStruct((H, NS, 640), jnp.float32),
        ],
        scratch_types=[
            pltpu.VMEM((2 * N,), jnp.float32),
            pltpu.VMEM((G, CHUNK), jnp.int32),
            pltpu.VMEM((G, CHUNK), jnp.int32),
            pltpu.VMEM((2, CHUNK, F), jnp.float32),
            pltpu.VMEM((2, CHUNK + 16), jnp.float32),
            pltpu.VMEM((640,), jnp.float32),
            pltpu.VMEM_SHARED((NPAD, F), jnp.float32),
            pltpu.VMEM_SHARED((NS * 640,), jnp.float32),
        ] + [pltpu.SemaphoreType.DMA] * 6,
    )(y2, a2, src4, dst4)
    acc = acc.reshape(H, NPAD, F)[:, :N]
    den2 = den.reshape(H, NS * 640)[:, :N]

    out = pl.pallas_call(
        _finish_body,
        out_shape=jax.ShapeDtypeStruct((N, F), jnp.float32),
    )(acc, den2, b_conv, W_lin, b_lin)
    return out
